# baseline (speedup 1.0000x reference)
import functools

import jax
import jax.numpy as jnp
from jax import lax
from jax.experimental import pallas as pl
from jax.experimental.pallas import tpu as pltpu
from jax.experimental.pallas import tpu_sc as plsc

B, N, C, D, STRIDE = 4, 8192, 768, 256, 2
ROWS = 64
LANES = 128
GR = B * ROWS
NOUT = N // STRIDE
TOT_OUT = B * NOUT
NC, NS, L = 2, 16, 16
NW = NC * NS
RPW = TOT_OUT // NW
K = 16
CHUNKS = RPW // K
CB = C // L
EW = 2 * RPW
ER = 2 * EW // 128


def _sc_gather_pool():
    mesh = plsc.VectorSubcoreMesh(core_axis_name="c", subcore_axis_name="s")

    @functools.partial(
        pl.kernel,
        mesh=mesh,
        out_type=[
            jax.ShapeDtypeStruct((TOT_OUT, C), jnp.float32),
            jax.ShapeDtypeStruct((TOT_OUT * 2 // 128, 128), jnp.float32),
        ],
        scratch_types=[
            pltpu.VMEM((2, 2 * K), jnp.int32),
            pltpu.VMEM((2, 2 * K, C), jnp.float32),
            pltpu.VMEM((2, K, C), jnp.float32),
            pltpu.VMEM((ER, 128), jnp.int32),
            pltpu.VMEM((ER, 128), jnp.float32),
            pltpu.VMEM((ER // 2, 128), jnp.float32),
            pltpu.SemaphoreType.DMA,
            pltpu.SemaphoreType.DMA,
            pltpu.SemaphoreType.DMA,
            pltpu.SemaphoreType.DMA,
            pltpu.SemaphoreType.DMA,
            pltpu.SemaphoreType.DMA,
            pltpu.SemaphoreType.DMA,
        ],
    )
    def k(x_hbm, idx_hbm, eidx_hbm, coords_hbm, xp_hbm, cp_hbm,
          idx_v, gx_v, out_v, eidx_v, gc_v, cp_v,
          sem_x0, sem_x1, sem_i0, sem_i1, sem_o0, sem_o1, sem_c):
        wid = lax.axis_index("s") * NC + lax.axis_index("c")
        w_base = wid * RPW
        sems_x = (sem_x0, sem_x1)
        sems_i = (sem_i0, sem_i1)
        sems_o = (sem_o0, sem_o1)

        def issue_idx(c, buf):
            return pltpu.async_copy(
                idx_hbm.at[pl.ds(2 * (w_base + c * K), 2 * K)],
                idx_v.at[buf], sems_i[buf])

        def issue_gather(c, buf):
            return pltpu.async_copy(x_hbm.at[idx_v.at[buf]], gx_v.at[buf],
                                    sems_x[buf])

        pltpu.sync_copy(eidx_hbm.at[pl.ds(ER * wid, ER)], eidx_v)
        for i in range(ER):
            pltpu.async_copy(coords_hbm.at[eidx_v.at[i]], gc_v.at[i], sem_c)

        issue_idx(0, 0).wait()
        issue_gather(0, 0)
        issue_idx(1, 1)

        def half(cur, buf):
            nxt = cur + 1
            pltpu.make_async_copy(x_hbm.at[idx_v.at[buf]], gx_v.at[buf],
                                  sems_x[buf]).wait()
            obuf = 1 - buf

            @pl.when(nxt < CHUNKS)
            def _():
                pltpu.make_async_copy(
                    idx_hbm.at[pl.ds(2 * (w_base + nxt * K), 2 * K)],
                    idx_v.at[obuf], sems_i[obuf]).wait()
                issue_gather(nxt, obuf)

            @pl.when(nxt + 1 < CHUNKS)
            def _():
                issue_idx(nxt + 1, buf)

            @pl.when(cur >= 2)
            def _():
                pltpu.make_async_copy(
                    out_v.at[buf],
                    xp_hbm.at[pl.ds(w_base + (cur - 2) * K, K)],
                    sems_o[buf]).wait()

            @plsc.parallel_loop(0, K, unroll=4)
            def _row(r):
                for cb in range(CB):
                    a = gx_v[buf, 2 * r, pl.ds(cb * L, L)]
                    bb = gx_v[buf, 2 * r + 1, pl.ds(cb * L, L)]
                    out_v[buf, r, pl.ds(cb * L, L)] = jnp.maximum(a, bb)
            pltpu.async_copy(out_v.at[buf],
                             xp_hbm.at[pl.ds(w_base + cur * K, K)],
                             sems_o[buf])

        def chunk_body(g, carry):
            half(2 * g, 0)
            half(2 * g + 1, 1)
            return carry

        lax.fori_loop(0, CHUNKS // 2, chunk_body, 0)

        pltpu.make_async_copy(
            out_v.at[0], xp_hbm.at[pl.ds(w_base + (CHUNKS - 2) * K, K)],
            sems_o[0]).wait()
        pltpu.make_async_copy(
            out_v.at[1], xp_hbm.at[pl.ds(w_base + (CHUNKS - 1) * K, K)],
            sems_o[1]).wait()

        for i in range(ER):
            pltpu.make_async_copy(coords_hbm.at[eidx_v.at[i]], gc_v.at[i],
                                  sem_c).wait()
        for r in range(ER // 2):
            for u in range(128 // L):
                a = gc_v[r, pl.ds(u * L, L)]
                bb = gc_v[r + ER // 2, pl.ds(u * L, L)]
                cp_v[r, pl.ds(u * L, L)] = (a + bb) * 0.5
        pltpu.sync_copy(cp_v, cp_hbm.at[pl.ds((ER // 2) * wid, ER // 2)])

    return k


_SC_CACHE = []


def _sc_kernel():
    if not _SC_CACHE:
        _SC_CACHE.append(_sc_gather_pool())
    return _SC_CACHE[0]


def _sort_body(eta_ref, gidx_ref, eidx_ref):
    u = lax.bitcast_convert_type(eta_ref[...], jnp.int32)
    u = jnp.where(u == jnp.int32(-2147483648), jnp.int32(0), u)
    key = jnp.where(u < 0, u ^ jnp.int32(0x7FFFFFFF), u)
    row = lax.broadcasted_iota(jnp.int32, (GR, LANES), 0)
    col = lax.broadcasted_iota(jnp.int32, (GR, LANES), 1)
    i2d = (row % ROWS) * LANES + col
    idx = i2d

    def partner(x, s):
        if s < LANES:
            r_l = jnp.concatenate([x[:, s:], x[:, :s]], axis=1)
            r_r = jnp.concatenate([x[:, LANES - s:], x[:, :LANES - s]], axis=1)
            return jnp.where((col & s) == 0, r_l, r_r)
        sr = s // LANES
        g = GR // (2 * sr)
        xr = x.reshape(g, 2, sr, LANES)
        xr = jnp.concatenate([xr[:, 1:2], xr[:, 0:1]], axis=1)
        return xr.reshape(GR, LANES)

    kk = 2
    while kk <= N:
        s = kk // 2
        while s >= 1:
            pk = partner(key, s)
            pi = partner(idx, s)
            gt = (key > pk) | ((key == pk) & (idx > pi))
            lower = (i2d & s) == 0
            asc = (i2d & kk) == 0
            keep = gt ^ (lower == asc)
            key = jnp.where(keep, key, pk)
            idx = jnp.where(keep, idx, pi)
            s //= 2
        kk *= 2
    g = idx + (row // ROWS) * N
    gidx_ref[...] = g

    even = (col & 1) == 0
    dbit = col & 1
    dup_even = jnp.where(even, g, jnp.concatenate([g[:, -1:], g[:, :-1]], 1))
    dup_odd = jnp.where(even, jnp.concatenate([g[:, 1:], g[:, :1]], 1), g)
    ea = 2 * dup_even + dbit
    eb = 2 * dup_odd + dbit
    packed = jnp.concatenate(
        [ea.reshape(NW, ER // 2, LANES), eb.reshape(NW, ER // 2, LANES)],
        axis=1).reshape(NW * ER, LANES)
    eidx_ref[...] = packed


def _argsort_tc(eta):
    gidx, eidx = pl.pallas_call(
        _sort_body,
        out_shape=(
            jax.ShapeDtypeStruct((GR, LANES), jnp.int32),
            jax.ShapeDtypeStruct((NW * ER, LANES), jnp.int32),
        ),
    )(eta.reshape(GR, LANES))
    return gidx.reshape(B * N), eidx


_BM = 4096


def _mm_ln_body(xp_ref, w_ref, b_ref, g_ref, be_ref, o_ref):
    h = jnp.dot(xp_ref[...], w_ref[...], preferred_element_type=jnp.float32)
    h = h + b_ref[...]
    mean = jnp.mean(h, axis=-1, keepdims=True)
    cen = h - mean
    var = jnp.mean(cen * cen, axis=-1, keepdims=True)
    hn = cen / jnp.sqrt(var + 1e-6)
    o_ref[...] = hn * g_ref[...] + be_ref[...]


def _mm_ln(xp, W, b, gamma, beta):
    grid = (TOT_OUT // _BM,)
    return pl.pallas_call(
        _mm_ln_body,
        grid=grid,
        in_specs=[
            pl.BlockSpec((_BM, C), lambda i: (i, 0)),
            pl.BlockSpec((C, D), lambda i: (0, 0)),
            pl.BlockSpec((1, D), lambda i: (0, 0)),
            pl.BlockSpec((1, D), lambda i: (0, 0)),
            pl.BlockSpec((1, D), lambda i: (0, 0)),
        ],
        out_specs=pl.BlockSpec((_BM, D), lambda i: (i, 0)),
        out_shape=jax.ShapeDtypeStruct((TOT_OUT, D), jnp.float32),
    )(xp, W, b.reshape(1, D), gamma.reshape(1, D), beta.reshape(1, D))


def kernel(x, coords, W, b, gamma, beta):
    eta = coords[..., 0]
    gidx, eidx = _argsort_tc(eta)
    x_flat = x.reshape(B * N, C)
    c_flat = coords.reshape(B * N * 2)
    xp, cp = _sc_kernel()(x_flat, gidx, eidx, c_flat)
    h = _mm_ln(xp, W, b, gamma, beta)
    return h.reshape(B, NOUT, D), cp.reshape(B, NOUT, 2)


# speedup vs baseline: 1.0177x; 1.0177x over previous
import functools

import jax
import jax.numpy as jnp
from jax import lax
from jax.experimental import pallas as pl
from jax.experimental.pallas import tpu as pltpu
from jax.experimental.pallas import tpu_sc as plsc

B, N, C, D, STRIDE = 4, 8192, 768, 256, 2
ROWS = 64
LANES = 128
GR = B * ROWS
NOUT = N // STRIDE
TOT_OUT = B * NOUT
NC, NS, L = 2, 16, 16
NW = NC * NS
RPW = TOT_OUT // NW
K = 16
CHUNKS = RPW // K
CB = C // L
EW = 2 * RPW
ER = 2 * EW // 128


def _sc_gather_pool():
    mesh = plsc.VectorSubcoreMesh(core_axis_name="c", subcore_axis_name="s")

    @functools.partial(
        pl.kernel,
        mesh=mesh,
        out_type=[
            jax.ShapeDtypeStruct((TOT_OUT, C), jnp.float32),
            jax.ShapeDtypeStruct((TOT_OUT * 2 // 128, 128), jnp.float32),
        ],
        scratch_types=[
            pltpu.VMEM((2, 2 * K), jnp.int32),
            pltpu.VMEM((2, 2 * K, C), jnp.float32),
            pltpu.VMEM((2, K, C), jnp.float32),
            pltpu.VMEM((ER, 128), jnp.int32),
            pltpu.VMEM((ER, 128), jnp.float32),
            pltpu.VMEM((ER // 2, 128), jnp.float32),
            pltpu.SemaphoreType.DMA,
            pltpu.SemaphoreType.DMA,
            pltpu.SemaphoreType.DMA,
            pltpu.SemaphoreType.DMA,
            pltpu.SemaphoreType.DMA,
            pltpu.SemaphoreType.DMA,
            pltpu.SemaphoreType.DMA,
        ],
    )
    def k(x_hbm, idx_hbm, eidx_hbm, coords_hbm, xp_hbm, cp_hbm,
          idx_v, gx_v, out_v, eidx_v, gc_v, cp_v,
          sem_x0, sem_x1, sem_i0, sem_i1, sem_o0, sem_o1, sem_c):
        wid = lax.axis_index("s") * NC + lax.axis_index("c")
        w_base = wid * RPW
        sems_x = (sem_x0, sem_x1)
        sems_i = (sem_i0, sem_i1)
        sems_o = (sem_o0, sem_o1)

        def issue_idx(c, buf):
            return pltpu.async_copy(
                idx_hbm.at[pl.ds(2 * (w_base + c * K), 2 * K)],
                idx_v.at[buf], sems_i[buf])

        def issue_gather(c, buf):
            return pltpu.async_copy(x_hbm.at[idx_v.at[buf]], gx_v.at[buf],
                                    sems_x[buf])

        pltpu.sync_copy(eidx_hbm.at[pl.ds(ER * wid, ER)], eidx_v)
        for i in range(ER):
            pltpu.async_copy(coords_hbm.at[eidx_v.at[i]], gc_v.at[i], sem_c)

        issue_idx(0, 0).wait()
        issue_gather(0, 0)
        issue_idx(1, 1)

        def half(cur, buf):
            nxt = cur + 1
            pltpu.make_async_copy(x_hbm.at[idx_v.at[buf]], gx_v.at[buf],
                                  sems_x[buf]).wait()
            obuf = 1 - buf

            @pl.when(nxt < CHUNKS)
            def _():
                pltpu.make_async_copy(
                    idx_hbm.at[pl.ds(2 * (w_base + nxt * K), 2 * K)],
                    idx_v.at[obuf], sems_i[obuf]).wait()
                issue_gather(nxt, obuf)

            @pl.when(nxt + 1 < CHUNKS)
            def _():
                issue_idx(nxt + 1, buf)

            @pl.when(cur >= 2)
            def _():
                pltpu.make_async_copy(
                    out_v.at[buf],
                    xp_hbm.at[pl.ds(w_base + (cur - 2) * K, K)],
                    sems_o[buf]).wait()

            @plsc.parallel_loop(0, K, unroll=2)
            def _row(r):
                for cb in range(CB):
                    a = gx_v[buf, 2 * r, pl.ds(cb * L, L)]
                    bb = gx_v[buf, 2 * r + 1, pl.ds(cb * L, L)]
                    out_v[buf, r, pl.ds(cb * L, L)] = jnp.maximum(a, bb)
            pltpu.async_copy(out_v.at[buf],
                             xp_hbm.at[pl.ds(w_base + cur * K, K)],
                             sems_o[buf])

        def chunk_body(g, carry):
            half(2 * g, 0)
            half(2 * g + 1, 1)
            return carry

        lax.fori_loop(0, CHUNKS // 2, chunk_body, 0)

        pltpu.make_async_copy(
            out_v.at[0], xp_hbm.at[pl.ds(w_base + (CHUNKS - 2) * K, K)],
            sems_o[0]).wait()
        pltpu.make_async_copy(
            out_v.at[1], xp_hbm.at[pl.ds(w_base + (CHUNKS - 1) * K, K)],
            sems_o[1]).wait()

        for i in range(ER):
            pltpu.make_async_copy(coords_hbm.at[eidx_v.at[i]], gc_v.at[i],
                                  sem_c).wait()
        for r in range(ER // 2):
            for u in range(128 // L):
                a = gc_v[r, pl.ds(u * L, L)]
                bb = gc_v[r + ER // 2, pl.ds(u * L, L)]
                cp_v[r, pl.ds(u * L, L)] = (a + bb) * 0.5
        pltpu.sync_copy(cp_v, cp_hbm.at[pl.ds((ER // 2) * wid, ER // 2)])

    return k


_SC_CACHE = []


def _sc_kernel():
    if not _SC_CACHE:
        _SC_CACHE.append(_sc_gather_pool())
    return _SC_CACHE[0]


def _sort_body(eta_ref, gidx_ref, eidx_ref):
    u = lax.bitcast_convert_type(eta_ref[...], jnp.int32)
    u = jnp.where(u == jnp.int32(-2147483648), jnp.int32(0), u)
    key = jnp.where(u < 0, u ^ jnp.int32(0x7FFFFFFF), u)
    row = lax.broadcasted_iota(jnp.int32, (GR, LANES), 0)
    col = lax.broadcasted_iota(jnp.int32, (GR, LANES), 1)
    i2d = (row % ROWS) * LANES + col
    idx = i2d

    def partner(x, s):
        if s < LANES:
            r_l = jnp.concatenate([x[:, s:], x[:, :s]], axis=1)
            r_r = jnp.concatenate([x[:, LANES - s:], x[:, :LANES - s]], axis=1)
            return jnp.where((col & s) == 0, r_l, r_r)
        sr = s // LANES
        g = GR // (2 * sr)
        xr = x.reshape(g, 2, sr, LANES)
        xr = jnp.concatenate([xr[:, 1:2], xr[:, 0:1]], axis=1)
        return xr.reshape(GR, LANES)

    kk = 2
    while kk <= N:
        s = kk // 2
        while s >= 1:
            pk = partner(key, s)
            pi = partner(idx, s)
            gt = (key > pk) | ((key == pk) & (idx > pi))
            lower = (i2d & s) == 0
            asc = (i2d & kk) == 0
            keep = gt ^ (lower == asc)
            key = jnp.where(keep, key, pk)
            idx = jnp.where(keep, idx, pi)
            s //= 2
        kk *= 2
    g = idx + (row // ROWS) * N
    gidx_ref[...] = g

    even = (col & 1) == 0
    dbit = col & 1
    dup_even = jnp.where(even, g, jnp.concatenate([g[:, -1:], g[:, :-1]], 1))
    dup_odd = jnp.where(even, jnp.concatenate([g[:, 1:], g[:, :1]], 1), g)
    ea = 2 * dup_even + dbit
    eb = 2 * dup_odd + dbit
    packed = jnp.concatenate(
        [ea.reshape(NW, ER // 2, LANES), eb.reshape(NW, ER // 2, LANES)],
        axis=1).reshape(NW * ER, LANES)
    eidx_ref[...] = packed


def _argsort_tc(eta):
    gidx, eidx = pl.pallas_call(
        _sort_body,
        out_shape=(
            jax.ShapeDtypeStruct((GR, LANES), jnp.int32),
            jax.ShapeDtypeStruct((NW * ER, LANES), jnp.int32),
        ),
    )(eta.reshape(GR, LANES))
    return gidx.reshape(B * N), eidx


_BM = 4096


def _mm_ln_body(xp_ref, w_ref, b_ref, g_ref, be_ref, o_ref):
    h = jnp.dot(xp_ref[...], w_ref[...], preferred_element_type=jnp.float32)
    h = h + b_ref[...]
    mean = jnp.mean(h, axis=-1, keepdims=True)
    cen = h - mean
    var = jnp.mean(cen * cen, axis=-1, keepdims=True)
    hn = cen / jnp.sqrt(var + 1e-6)
    o_ref[...] = hn * g_ref[...] + be_ref[...]


def _mm_ln(xp, W, b, gamma, beta):
    grid = (TOT_OUT // _BM,)
    return pl.pallas_call(
        _mm_ln_body,
        grid=grid,
        in_specs=[
            pl.BlockSpec((_BM, C), lambda i: (i, 0)),
            pl.BlockSpec((C, D), lambda i: (0, 0)),
            pl.BlockSpec((1, D), lambda i: (0, 0)),
            pl.BlockSpec((1, D), lambda i: (0, 0)),
            pl.BlockSpec((1, D), lambda i: (0, 0)),
        ],
        out_specs=pl.BlockSpec((_BM, D), lambda i: (i, 0)),
        out_shape=jax.ShapeDtypeStruct((TOT_OUT, D), jnp.float32),
    )(xp, W, b.reshape(1, D), gamma.reshape(1, D), beta.reshape(1, D))


def kernel(x, coords, W, b, gamma, beta):
    eta = coords[..., 0]
    gidx, eidx = _argsort_tc(eta)
    x_flat = x.reshape(B * N, C)
    c_flat = coords.reshape(B * N * 2)
    xp, cp = _sc_kernel()(x_flat, gidx, eidx, c_flat)
    h = _mm_ln(xp, W, b, gamma, beta)
    return h.reshape(B, NOUT, D), cp.reshape(B, NOUT, 2)
